# baseline (speedup 1.0000x reference)
import functools
import math

import numpy as np
import jax
import jax.numpy as jnp
from jax import lax
from jax.experimental import pallas as pl
from jax.experimental.pallas import tpu as pltpu
from jax.experimental.pallas import tpu_sc as plsc

DIM = 1024
NSUB = 512
KDIM = 256
CDIM = 256
TSUB = 32
TFIN = 32
SCALE = 1.0 / math.sqrt(2.0 * KDIM)

BLK_A = 512
BLK_C = 512
NEG = -1e30

_PAIRS = [(i, j) for i in range(TSUB) for j in range(TSUB)
          if (i + 1) * (j + 1) <= TFIN]
P_CAND = 128
_K1 = np.zeros((TSUB, P_CAND), np.float32)
_K2 = np.zeros((TSUB, P_CAND), np.float32)
_BIAS = np.full((1, P_CAND), NEG, np.float32)
for _p, (_i, _j) in enumerate(_PAIRS):
    _K1[_i, _p] = 1.0
    _K2[_j, _p] = 1.0
    _BIAS[0, _p] = 0.0


def _topk_desc(s, k, payload=None):
    R, C = s.shape
    iota = lax.broadcasted_iota(jnp.int32, (R, C), 1)
    outs_s, outs_p = [], []
    for _ in range(k):
        m = jnp.max(s, axis=1, keepdims=True)
        pos = jnp.argmax(s, axis=1, keepdims=True).astype(jnp.int32)
        sel = iota == pos
        outs_s.append(m)
        if payload is None:
            outs_p.append(pos.astype(jnp.float32))
        else:
            outs_p.append(jnp.sum(jnp.where(sel, payload, jnp.float32(0)),
                                  axis=1, keepdims=True))
        s = jnp.where(sel, jnp.float32(NEG), s)
    return jnp.concatenate(outs_s, 1), jnp.concatenate(outs_p, 1)


def _select_body(x_ref, wq_ref, ka_ref, kb_ref, k1_ref, k2_ref, bias_ref,
                 fi_ref, w_ref, ssum_ref, smax_ref):
    q = jnp.dot(x_ref[...], wq_ref[...], preferred_element_type=jnp.float32)
    dn = (((1,), (1,)), ((), ()))
    sa = lax.dot_general(q[:, :KDIM], ka_ref[...], dn,
                         preferred_element_type=jnp.float32)
    sb = lax.dot_general(q[:, KDIM:], kb_ref[...], dn,
                         preferred_element_type=jnp.float32)
    sab = jnp.concatenate([sa, sb], axis=0)
    t_s, t_i = _topk_desc(sab, TSUB)
    B = sa.shape[0]
    ta_s, tb_s = t_s[:B], t_s[B:]
    ta_i, tb_i = t_i[:B], t_i[B:]
    k1 = k1_ref[...]
    k2 = k2_ref[...]
    cs = (jnp.dot(ta_s, k1, preferred_element_type=jnp.float32)
          + jnp.dot(tb_s, k2, preferred_element_type=jnp.float32)
          + bias_ref[...])
    ci = (jnp.dot(ta_i, k1, preferred_element_type=jnp.float32)
          * jnp.float32(NSUB)
          + jnp.dot(tb_i, k2, preferred_element_type=jnp.float32))
    fs, fi = _topk_desc(cs, TFIN, payload=ci)
    fi_ref[...] = fi.astype(jnp.int32)
    e = jnp.exp((fs - fs[:, :1]) * jnp.float32(SCALE))
    w_ref[...] = e / jnp.sum(e, axis=1, keepdims=True)

    @pl.when(pl.program_id(0) == 0)
    def _init():
        ssum_ref[0, 0] = jnp.float32(0)
        smax_ref[0, 0] = jnp.float32(NEG)

    ssum_ref[0, 0] += jnp.sum(fs)
    smax_ref[0, 0] = jnp.maximum(smax_ref[0, 0], jnp.max(fs))


def _mlp_body(mix_ref, w1_ref, w2_ref, out_ref):
    h = jnp.dot(mix_ref[...], w1_ref[...], preferred_element_type=jnp.float32)
    h = h * lax.logistic(h)
    out_ref[...] = jnp.dot(h, w2_ref[...], preferred_element_type=jnp.float32)


def _select_call(xf, W_q, key_a, key_b, n_tok):
    k1 = jnp.asarray(_K1)
    k2 = jnp.asarray(_K2)
    bias = jnp.asarray(_BIAS)
    grid = n_tok // BLK_A
    return pl.pallas_call(
        _select_body,
        grid=(grid,),
        in_specs=[
            pl.BlockSpec((BLK_A, DIM), lambda i: (i, 0)),
            pl.BlockSpec((DIM, 2 * KDIM), lambda i: (0, 0)),
            pl.BlockSpec((NSUB, KDIM), lambda i: (0, 0)),
            pl.BlockSpec((NSUB, KDIM), lambda i: (0, 0)),
            pl.BlockSpec((TSUB, P_CAND), lambda i: (0, 0)),
            pl.BlockSpec((TSUB, P_CAND), lambda i: (0, 0)),
            pl.BlockSpec((1, P_CAND), lambda i: (0, 0)),
        ],
        out_specs=[
            pl.BlockSpec((BLK_A, TFIN), lambda i: (i, 0)),
            pl.BlockSpec((BLK_A, TFIN), lambda i: (i, 0)),
            pl.BlockSpec(block_shape=(1, 1), index_map=lambda i: (0, 0),
                         memory_space=pltpu.SMEM),
            pl.BlockSpec(block_shape=(1, 1), index_map=lambda i: (0, 0),
                         memory_space=pltpu.SMEM),
        ],
        out_shape=[
            jax.ShapeDtypeStruct((n_tok, TFIN), jnp.int32),
            jax.ShapeDtypeStruct((n_tok, TFIN), jnp.float32),
            jax.ShapeDtypeStruct((1, 1), jnp.float32),
            jax.ShapeDtypeStruct((1, 1), jnp.float32),
        ],
        compiler_params=pltpu.CompilerParams(
            dimension_semantics=("arbitrary",)),
    )(xf, W_q, key_a, key_b, k1, k2, bias)


def _gather_mix_call(idx2, w, codes, n_tok):
    NC, NS = 2, 16
    NW = NC * NS
    tok_w = n_tok // NW
    GRP = 4
    rows_g = GRP * TFIN
    ngrp = tok_w // GRP
    mesh = plsc.VectorSubcoreMesh(core_axis_name="c", subcore_axis_name="s")

    @functools.partial(
        pl.kernel, mesh=mesh,
        out_type=jax.ShapeDtypeStruct((n_tok, CDIM), jnp.float32),
        scratch_types=[
            pltpu.VMEM((ngrp, rows_g), jnp.int32),
            pltpu.VMEM((tok_w, TFIN), jnp.float32),
            pltpu.VMEM((2, rows_g, CDIM), jnp.float32),
            pltpu.VMEM((GRP, CDIM), jnp.float32),
            pltpu.SemaphoreType.DMA,
            pltpu.SemaphoreType.DMA,
        ],
    )
    def _sc(idx_hbm, w_hbm, codes_hbm, out_hbm, idx_v, w_v, rows_v, acc_v,
            sem0, sem1):
        wid = lax.axis_index("s") * NC + lax.axis_index("c")
        tok0 = wid * tok_w
        pltpu.sync_copy(idx_hbm.at[pl.ds(wid * ngrp, ngrp)], idx_v)
        pltpu.sync_copy(w_hbm.at[pl.ds(tok0, tok_w)], w_v)
        sems = (sem0, sem1)

        def gather(g, b):
            return pltpu.make_async_copy(
                codes_hbm.at[idx_v.at[g]], rows_v.at[b], sems[b])

        gather(0, 0).start()
        gather(1, 1).start()

        def outer(it, carry):
            g0 = it * 2
            for b in range(2):
                g = g0 + b
                gather(g, b).wait()
                rv = rows_v.at[b]

                def tbody(tt, c, g=g, rv=rv):
                    wrow = g * GRP + tt
                    acc = [None] * (CDIM // 16)
                    first = True
                    for jc in range(TFIN // 16):
                        wv = w_v[wrow, pl.ds(jc * 16, 16)]
                        for jj in range(16):
                            wgt = wv[jj]
                            row = tt * TFIN + jc * 16 + jj
                            for d in range(CDIM // 16):
                                v = rv[row, pl.ds(d * 16, 16)] * wgt
                                acc[d] = v if first else acc[d] + v
                            first = False
                    for d in range(CDIM // 16):
                        acc_v[tt, pl.ds(d * 16, 16)] = acc[d]
                    return c

                lax.fori_loop(0, GRP, tbody, 0)

                @pl.when(g + 2 < ngrp)
                def _():
                    gather(g + 2, b).start()

                pltpu.sync_copy(acc_v,
                                out_hbm.at[pl.ds(tok0 + g * GRP, GRP)])
            return carry

        lax.fori_loop(0, ngrp // 2, outer, 0)

    return _sc(idx2, w, codes)


def _mlp_call(mixed, W1, W2, n_tok):
    grid = n_tok // BLK_C
    return pl.pallas_call(
        _mlp_body,
        grid=(grid,),
        in_specs=[
            pl.BlockSpec((BLK_C, CDIM), lambda i: (i, 0)),
            pl.BlockSpec((CDIM, DIM), lambda i: (0, 0)),
            pl.BlockSpec((DIM, DIM), lambda i: (0, 0)),
        ],
        out_specs=pl.BlockSpec((BLK_C, DIM), lambda i: (i, 0)),
        out_shape=jax.ShapeDtypeStruct((n_tok, DIM), jnp.float32),
    )(mixed, W1, W2)


def kernel(x, W_q, key_a, key_b, codes, W1, W2):
    batch, seq, _ = x.shape
    n_tok = batch * seq
    xf = x.reshape(n_tok, DIM)
    H = 4
    chunk = n_tok // H
    outs, ssums, smaxs = [], [], []
    for h in range(H):
        xf_h = lax.slice_in_dim(xf, h * chunk, (h + 1) * chunk, axis=0)
        fi, w, ssum, smax = _select_call(xf_h, W_q, key_a, key_b, chunk)
        idx2 = fi.reshape(chunk * TFIN // 128, 128)
        mixed = _gather_mix_call(idx2, w, codes, chunk)
        outs.append(_mlp_call(mixed, W1, W2, chunk))
        ssums.append(ssum[0, 0])
        smaxs.append(smax[0, 0])
    out = jnp.concatenate(outs, axis=0)
    stats_mean = sum(ssums) / jnp.float32(n_tok * TFIN)
    stats_max = jnp.stack(smaxs).max()
    return (out.reshape(batch, seq, DIM), stats_mean, stats_max)


# speedup vs baseline: 1.5731x; 1.5731x over previous
import functools
import math

import numpy as np
import jax
import jax.numpy as jnp
from jax import lax
from jax.experimental import pallas as pl
from jax.experimental.pallas import tpu as pltpu
from jax.experimental.pallas import tpu_sc as plsc

DIM = 1024
NSUB = 512
KDIM = 256
CDIM = 256
TSUB = 32
TFIN = 32
SCALE = 1.0 / math.sqrt(2.0 * KDIM)

BLK_A = 512
BLK_C = 512
NEG = -1e30

_PAIRS = [(i, j) for i in range(TSUB) for j in range(TSUB)
          if (i + 1) * (j + 1) <= TFIN]
P_CAND = 128
_K1 = np.zeros((TSUB, P_CAND), np.float32)
_K2 = np.zeros((TSUB, P_CAND), np.float32)
_BIAS = np.full((1, P_CAND), NEG, np.float32)
for _p, (_i, _j) in enumerate(_PAIRS):
    _K1[_i, _p] = 1.0
    _K2[_j, _p] = 1.0
    _BIAS[0, _p] = 0.0


def _topk_desc(s, k, payload=None):
    R, C = s.shape
    iota = lax.broadcasted_iota(jnp.int32, (R, C), 1)
    big = jnp.int32(C)
    outs_s, outs_p = [], []
    for _ in range(k):
        m = jnp.max(s, axis=1, keepdims=True)
        hit = s == m
        pos = jnp.min(jnp.where(hit, iota, big), axis=1, keepdims=True)
        sel = iota == pos
        outs_s.append(m)
        if payload is None:
            outs_p.append(pos.astype(jnp.float32))
        else:
            outs_p.append(jnp.sum(jnp.where(sel, payload, jnp.float32(0)),
                                  axis=1, keepdims=True))
        s = jnp.where(sel, jnp.float32(NEG), s)
    return jnp.concatenate(outs_s, 1), jnp.concatenate(outs_p, 1)


def _select_body(x_ref, wq_ref, ka_ref, kb_ref, k1_ref, k2_ref, bias_ref,
                 fi_ref, w_ref, ssum_ref, smax_ref):
    q = jnp.dot(x_ref[...], wq_ref[...], preferred_element_type=jnp.float32)
    dn = (((1,), (1,)), ((), ()))
    sa = lax.dot_general(q[:, :KDIM], ka_ref[...], dn,
                         preferred_element_type=jnp.float32)
    sb = lax.dot_general(q[:, KDIM:], kb_ref[...], dn,
                         preferred_element_type=jnp.float32)
    sab = jnp.concatenate([sa, sb], axis=0)
    t_s, t_i = _topk_desc(sab, TSUB)
    B = sa.shape[0]
    ta_s, tb_s = t_s[:B], t_s[B:]
    ta_i, tb_i = t_i[:B], t_i[B:]
    k1 = k1_ref[...]
    k2 = k2_ref[...]
    cs = (jnp.dot(ta_s, k1, preferred_element_type=jnp.float32)
          + jnp.dot(tb_s, k2, preferred_element_type=jnp.float32)
          + bias_ref[...])
    ci = (jnp.dot(ta_i, k1, preferred_element_type=jnp.float32)
          * jnp.float32(NSUB)
          + jnp.dot(tb_i, k2, preferred_element_type=jnp.float32))
    fs, fi = _topk_desc(cs, TFIN, payload=ci)
    fi_ref[...] = fi.astype(jnp.int32)
    e = jnp.exp((fs - fs[:, :1]) * jnp.float32(SCALE))
    w_ref[...] = e / jnp.sum(e, axis=1, keepdims=True)

    @pl.when(pl.program_id(0) == 0)
    def _init():
        ssum_ref[0, 0] = jnp.float32(0)
        smax_ref[0, 0] = jnp.float32(NEG)

    ssum_ref[0, 0] += jnp.sum(fs)
    smax_ref[0, 0] = jnp.maximum(smax_ref[0, 0], jnp.max(fs))


def _mlp_body(mix_ref, w1_ref, w2_ref, out_ref):
    h = jnp.dot(mix_ref[...], w1_ref[...], preferred_element_type=jnp.float32)
    h = h * lax.logistic(h)
    out_ref[...] = jnp.dot(h, w2_ref[...], preferred_element_type=jnp.float32)


def _select_call(xf, W_q, key_a, key_b, n_tok):
    k1 = jnp.asarray(_K1)
    k2 = jnp.asarray(_K2)
    bias = jnp.asarray(_BIAS)
    grid = n_tok // BLK_A
    return pl.pallas_call(
        _select_body,
        grid=(grid,),
        in_specs=[
            pl.BlockSpec((BLK_A, DIM), lambda i: (i, 0)),
            pl.BlockSpec((DIM, 2 * KDIM), lambda i: (0, 0)),
            pl.BlockSpec((NSUB, KDIM), lambda i: (0, 0)),
            pl.BlockSpec((NSUB, KDIM), lambda i: (0, 0)),
            pl.BlockSpec((TSUB, P_CAND), lambda i: (0, 0)),
            pl.BlockSpec((TSUB, P_CAND), lambda i: (0, 0)),
            pl.BlockSpec((1, P_CAND), lambda i: (0, 0)),
        ],
        out_specs=[
            pl.BlockSpec((BLK_A, TFIN), lambda i: (i, 0)),
            pl.BlockSpec((BLK_A, TFIN), lambda i: (i, 0)),
            pl.BlockSpec(block_shape=(1, 1), index_map=lambda i: (0, 0),
                         memory_space=pltpu.SMEM),
            pl.BlockSpec(block_shape=(1, 1), index_map=lambda i: (0, 0),
                         memory_space=pltpu.SMEM),
        ],
        out_shape=[
            jax.ShapeDtypeStruct((n_tok, TFIN), jnp.int32),
            jax.ShapeDtypeStruct((n_tok, TFIN), jnp.float32),
            jax.ShapeDtypeStruct((1, 1), jnp.float32),
            jax.ShapeDtypeStruct((1, 1), jnp.float32),
        ],
        compiler_params=pltpu.CompilerParams(
            dimension_semantics=("arbitrary",)),
    )(xf, W_q, key_a, key_b, k1, k2, bias)


def _gather_mix_call(idx2, w, codes, n_tok):
    NC, NS = 2, 16
    NW = NC * NS
    tok_w = n_tok // NW
    GRP = 4
    rows_g = GRP * TFIN
    ngrp = tok_w // GRP
    mesh = plsc.VectorSubcoreMesh(core_axis_name="c", subcore_axis_name="s")

    @functools.partial(
        pl.kernel, mesh=mesh,
        out_type=jax.ShapeDtypeStruct((n_tok, CDIM), jnp.float32),
        scratch_types=[
            pltpu.VMEM((ngrp, rows_g), jnp.int32),
            pltpu.VMEM((tok_w, TFIN), jnp.float32),
            pltpu.VMEM((2, rows_g, CDIM), jnp.float32),
            pltpu.VMEM((GRP, CDIM), jnp.float32),
            pltpu.SemaphoreType.DMA,
            pltpu.SemaphoreType.DMA,
        ],
    )
    def _sc(idx_hbm, w_hbm, codes_hbm, out_hbm, idx_v, w_v, rows_v, acc_v,
            sem0, sem1):
        wid = lax.axis_index("s") * NC + lax.axis_index("c")
        tok0 = wid * tok_w
        pltpu.sync_copy(idx_hbm.at[pl.ds(wid * ngrp, ngrp)], idx_v)
        pltpu.sync_copy(w_hbm.at[pl.ds(tok0, tok_w)], w_v)
        sems = (sem0, sem1)

        def gather(g, b):
            return pltpu.make_async_copy(
                codes_hbm.at[idx_v.at[g]], rows_v.at[b], sems[b])

        gather(0, 0).start()
        gather(1, 1).start()

        def outer(it, carry):
            g0 = it * 2
            for b in range(2):
                g = g0 + b
                gather(g, b).wait()
                rv = rows_v.at[b]

                def tbody(tt, c, g=g, rv=rv):
                    wrow = g * GRP + tt
                    acc = [None] * (CDIM // 16)
                    first = True
                    for jc in range(TFIN // 16):
                        wv = w_v[wrow, pl.ds(jc * 16, 16)]
                        for jj in range(16):
                            wgt = wv[jj]
                            row = tt * TFIN + jc * 16 + jj
                            for d in range(CDIM // 16):
                                v = rv[row, pl.ds(d * 16, 16)] * wgt
                                acc[d] = v if first else acc[d] + v
                            first = False
                    for d in range(CDIM // 16):
                        acc_v[tt, pl.ds(d * 16, 16)] = acc[d]
                    return c

                lax.fori_loop(0, GRP, tbody, 0)

                @pl.when(g + 2 < ngrp)
                def _():
                    gather(g + 2, b).start()

                pltpu.sync_copy(acc_v,
                                out_hbm.at[pl.ds(tok0 + g * GRP, GRP)])
            return carry

        lax.fori_loop(0, ngrp // 2, outer, 0)

    return _sc(idx2, w, codes)


def _mlp_call(mixed, W1, W2, n_tok):
    grid = n_tok // BLK_C
    return pl.pallas_call(
        _mlp_body,
        grid=(grid,),
        in_specs=[
            pl.BlockSpec((BLK_C, CDIM), lambda i: (i, 0)),
            pl.BlockSpec((CDIM, DIM), lambda i: (0, 0)),
            pl.BlockSpec((DIM, DIM), lambda i: (0, 0)),
        ],
        out_specs=pl.BlockSpec((BLK_C, DIM), lambda i: (i, 0)),
        out_shape=jax.ShapeDtypeStruct((n_tok, DIM), jnp.float32),
    )(mixed, W1, W2)


def kernel(x, W_q, key_a, key_b, codes, W1, W2):
    batch, seq, _ = x.shape
    n_tok = batch * seq
    xf = x.reshape(n_tok, DIM)
    H = 4
    chunk = n_tok // H
    outs, ssums, smaxs = [], [], []
    for h in range(H):
        xf_h = lax.slice_in_dim(xf, h * chunk, (h + 1) * chunk, axis=0)
        fi, w, ssum, smax = _select_call(xf_h, W_q, key_a, key_b, chunk)
        idx2 = fi.reshape(chunk * TFIN // 128, 128)
        mixed = _gather_mix_call(idx2, w, codes, chunk)
        outs.append(_mlp_call(mixed, W1, W2, chunk))
        ssums.append(ssum[0, 0])
        smaxs.append(smax[0, 0])
    out = jnp.concatenate(outs, axis=0)
    stats_mean = sum(ssums) / jnp.float32(n_tok * TFIN)
    stats_max = jnp.stack(smaxs).max()
    return (out.reshape(batch, seq, DIM), stats_mean, stats_max)
